# baseline (speedup 1.0000x reference)
import functools

import jax
import jax.numpy as jnp
import numpy as np
from jax import lax
from jax.experimental import pallas as pl
from jax.experimental.pallas import tpu as pltpu
from jax.experimental.pallas import tpu_sc as plsc

DIM = 64
SZ = 512
CHY = 8
XPADP = 520
YPAD = 520
P = 128

def _pack32(tlo, thi):
    ulo = lax.bitcast_convert_type(tlo, jnp.uint32) >> 16
    uhi = lax.bitcast_convert_type(thi, jnp.uint32) & jnp.uint32(0xFFFF0000)
    return lax.bitcast_convert_type(ulo | uhi, jnp.int32)


def _tables_body(xy_ref, xz_ref, yz_ref, wlo_ref, whi_ref,
                 o0_ref, o1_ref, o2_ref, carry_ref):
    j = pl.program_id(0)

    @pl.when(j == 0)
    def _():
        carry_ref[...] = jnp.zeros((3, 2, SZ, 32), jnp.float32)

    live = (j <= SZ // CHY - 1)
    for p, (ref, out_ref) in enumerate(((xy_ref, o0_ref), (xz_ref, o1_ref),
                                        (yz_ref, o2_ref))):
        a = ref[...]
        halves = []
        for h, w_ref in enumerate((wlo_ref, whi_ref)):
            wp = w_ref[:, p * DIM:(p + 1) * DIM]
            t = lax.dot_general(a, wp, (((0,), (1,)), ((), ())),
                                preferred_element_type=jnp.float32)
            t = jnp.where(live, t, 0.0)
            prev = carry_ref[p, h]
            halves.append((jnp.concatenate([prev[None], t[:-1]], 0), t))
            carry_ref[p, h] = t[-1]
        (lo_m1, lo), (hi_m1, hi) = halves
        pk0 = _pack32(lo_m1, hi_m1)
        pk1 = _pack32(lo, hi)
        zrow = jnp.zeros((XPADP - SZ, DIM), jnp.int32)
        for r in range(CHY):
            out_ref[r * XPADP:r * XPADP + SZ, 0:32] = pk0[r]
            out_ref[r * XPADP:r * XPADP + SZ, 32:64] = pk1[r]
            out_ref[r * XPADP + SZ:(r + 1) * XPADP, :] = zrow


def _build_tables(xy, xz, yz, wlo, whi):
    grid = YPAD // CHY
    last = SZ // CHY - 1
    bs_plane = pl.BlockSpec((DIM, CHY, SZ),
                            lambda j: (0, jnp.minimum(j, last), 0))
    bs_w = pl.BlockSpec((32, 3 * DIM), lambda j: (0, 0))
    bs_out = pl.BlockSpec((CHY * XPADP, DIM), lambda j: (j, 0))
    tab = jax.ShapeDtypeStruct((YPAD * XPADP, DIM), jnp.int32)
    return pl.pallas_call(
        _tables_body,
        grid=(grid,),
        in_specs=[bs_plane, bs_plane, bs_plane, bs_w, bs_w],
        out_specs=[bs_out, bs_out, bs_out],
        out_shape=[tab, tab, tab],
        scratch_shapes=[pltpu.VMEM((3, 2, SZ, 32), jnp.float32)],
    )(xy, xz, yz, wlo, whi)


def _axis_setup(g):
    ixf = ((g + 1.0) * float(SZ) - 1.0) * 0.5
    ixf = jnp.clip(ixf, -1.0, float(SZ))
    t = ixf.astype(jnp.int32)
    tf = t.astype(jnp.float32)
    c0 = jnp.where(tf > ixf, t - 1, t)
    w1 = ixf - c0.astype(jnp.float32)
    return c0, 1.0 - w1, w1


def _sc_sample(gx, gy, gz, tabs, bias):
    n = gx.shape[0]
    info = plsc.get_sparse_core_info()
    nw = info.num_cores * info.num_subcores
    npw = n // nw
    nblk = npw // P
    mesh = plsc.VectorSubcoreMesh(core_axis_name="c", subcore_axis_name="s")

    @functools.partial(
        pl.kernel, mesh=mesh,
        out_type=jax.ShapeDtypeStruct((n, DIM), jnp.float32),
        compiler_params=pltpu.CompilerParams(use_tc_tiling_on_sc=False,
                                             needs_layout_passes=False),
        scratch_types=[
            pltpu.VMEM((P,), jnp.float32),
            pltpu.VMEM((P,), jnp.float32),
            pltpu.VMEM((P,), jnp.float32),
            pltpu.VMEM((2, 6, P), jnp.int32),
            pltpu.VMEM((2, 12, P), jnp.float32),
            pltpu.VMEM((6, P, DIM), jnp.int32),
            pltpu.VMEM((6, P, DIM), jnp.int32),
            pltpu.VMEM((P, DIM), jnp.float32),
            pltpu.VMEM((P, DIM), jnp.float32),
            pltpu.VMEM((DIM,), jnp.float32),
            pltpu.SemaphoreType.DMA,
            pltpu.SemaphoreType.DMA,
            pltpu.SemaphoreType.DMA,
            pltpu.SemaphoreType.DMA,
        ],
    )
    def body(gx_h, gy_h, gz_h, t0_h, t1_h, t2_h, b_h, out_h,
             gx_v, gy_v, gz_v, idx_v, w_v, rows0_v, rows1_v, acc0_v, acc1_v,
             b_v, gsem0, gsem1, osem0, osem1):
        tab_hs = (t0_h, t1_h, t2_h)
        rows_vs = (rows0_v, rows1_v)
        acc_vs = (acc0_v, acc1_v)
        gsems = (gsem0, gsem1)
        osems = (osem0, osem1)
        wid = lax.axis_index("s") * info.num_cores + lax.axis_index("c")
        base0 = wid * npw
        pltpu.sync_copy(b_h, b_v)
        bias_regs = [b_v[pl.ds(k * 16, 16)] for k in range(4)]

        def stage_a(blk, buf):
            base = base0 + blk * P
            pltpu.sync_copy(gx_h.at[pl.ds(base, P)], gx_v)
            pltpu.sync_copy(gy_h.at[pl.ds(base, P)], gy_v)
            pltpu.sync_copy(gz_h.at[pl.ds(base, P)], gz_v)

            def grp_body(i, c2):
                s = pl.ds(i * 16, 16)
                ax_ = _axis_setup(gx_v[s])
                ay_ = _axis_setup(gy_v[s])
                az_ = _axis_setup(gz_v[s])
                for p, (axA, axB) in enumerate(((ax_, ay_), (ax_, az_), (ay_, az_))):
                    a0, aw0, aw1 = axA
                    b0, bw0, bw1 = axB
                    kx0 = jnp.where(a0 < 0, SZ + 2, a0)
                    rb = (b0 + 1) * XPADP
                    idx_v[buf, 2 * p, s] = rb + kx0
                    idx_v[buf, 2 * p + 1, s] = rb + a0 + 1
                    w_v[buf, 4 * p + 0, s] = aw0 * bw0
                    w_v[buf, 4 * p + 1, s] = aw0 * bw1
                    w_v[buf, 4 * p + 2, s] = aw1 * bw0
                    w_v[buf, 4 * p + 3, s] = aw1 * bw1
                return c2
            lax.fori_loop(0, P // 16, grp_body, 0)
            for r in range(6):
                pltpu.async_copy(tab_hs[r // 2].at[idx_v.at[buf, r]],
                                 rows_vs[buf].at[r], gsems[buf])

        def wait_gathers(buf):
            for r in range(6):
                pltpu.make_async_copy(tab_hs[r // 2].at[idx_v.at[buf, r]],
                                      rows_vs[buf].at[r], gsems[buf]).wait()

        def drain_out(buf):
            pltpu.make_async_copy(acc_vs[buf], out_h.at[pl.ds(base0, P)],
                                  osems[buf]).wait()

        def stage_b(blk, buf):
            rows_v = rows_vs[buf]
            acc_v = acc_vs[buf]

            def ptg_body(g, c2):
                sg = pl.ds(g * 16, 16)
                wvecs = [w_v[buf, c, sg] for c in range(12)]
                rbase = g * 16
                for j in range(16):
                    m = rbase + j
                    accs = list(bias_regs)
                    for r in range(6):
                        p, xc = r // 2, r % 2
                        for yc in range(2):
                            w = jnp.full((16,), wvecs[4 * p + 2 * xc + yc][j],
                                         jnp.float32)
                            for q in range(2):
                                v = rows_v[r, m, pl.ds(yc * 32 + q * 16, 16)]
                                even = plsc.bitcast(v << 16, jnp.float32)
                                odd = plsc.bitcast(v, jnp.float32)
                                accs[2 * q] = accs[2 * q] + w * even
                                accs[2 * q + 1] = accs[2 * q + 1] + w * odd
                    for k in range(4):
                        acc_v[m, pl.ds(k * 16, 16)] = accs[k]
                return c2
            lax.fori_loop(0, P // 16, ptg_body, 0)
            pltpu.async_copy(acc_v, out_h.at[pl.ds(base0 + blk * P, P)],
                             osems[buf])

        stage_a(0, 0)
        stage_a(1, 1)

        def pair_body(jp, carry):
            for buf in range(2):
                blk = 2 * jp + buf
                wait_gathers(buf)

                @pl.when(jp > 0)
                def _():
                    drain_out(buf)
                stage_b(blk, buf)

                @pl.when(jp < nblk // 2 - 1)
                def _():
                    stage_a(blk + 2, buf)
            return carry
        lax.fori_loop(0, nblk // 2, pair_body, 0)
        drain_out(0)
        drain_out(1)

    return body(gx, gy, gz, *tabs, bias)


def kernel(x, xy, xz, yz, lin_w, lin_b):
    wlo = jnp.concatenate([lin_w[0:16], lin_w[32:48]], 0)
    whi = jnp.concatenate([lin_w[16:32], lin_w[48:64]], 0)
    tabs = _build_tables(xy, xz, yz, wlo, whi)
    gx, gy, gz = x[:, 0], x[:, 1], x[:, 2]
    return _sc_sample(gx, gy, gz, tabs, lin_b)


# speedup vs baseline: 1.1258x; 1.1258x over previous
import functools

import jax
import jax.numpy as jnp
import numpy as np
from jax import lax
from jax.experimental import pallas as pl
from jax.experimental.pallas import tpu as pltpu
from jax.experimental.pallas import tpu_sc as plsc

DIM = 64
SZ = 512
CHY = 8
XPADP = 520
YPAD = 520
P = 128

def _pack32(tlo, thi):
    ulo = lax.bitcast_convert_type(tlo, jnp.uint32) >> 16
    uhi = lax.bitcast_convert_type(thi, jnp.uint32) & jnp.uint32(0xFFFF0000)
    return lax.bitcast_convert_type(ulo | uhi, jnp.int32)


def _tables_body(xy_ref, xz_ref, yz_ref, wlo_ref, whi_ref,
                 o0_ref, o1_ref, o2_ref, carry_ref):
    j = pl.program_id(0)

    @pl.when(j == 0)
    def _():
        carry_ref[...] = jnp.zeros((3, 2, SZ, 32), jnp.float32)

    live = (j <= SZ // CHY - 1)
    for p, (ref, out_ref) in enumerate(((xy_ref, o0_ref), (xz_ref, o1_ref),
                                        (yz_ref, o2_ref))):
        a = ref[...]
        halves = []
        for h, w_ref in enumerate((wlo_ref, whi_ref)):
            wp = w_ref[:, p * DIM:(p + 1) * DIM]
            t = lax.dot_general(a, wp, (((0,), (1,)), ((), ())),
                                preferred_element_type=jnp.float32)
            t = jnp.where(live, t, 0.0)
            prev = carry_ref[p, h]
            halves.append((jnp.concatenate([prev[None], t[:-1]], 0), t))
            carry_ref[p, h] = t[-1]
        (lo_m1, lo), (hi_m1, hi) = halves
        pk0 = _pack32(lo_m1, hi_m1)
        pk1 = _pack32(lo, hi)
        zpad = jnp.zeros((XPADP - SZ, 2 * DIM), jnp.int32)
        zrow = jnp.zeros((1, 2 * DIM), jnp.int32)
        for r in range(CHY):
            b = r * XPADP
            out_ref[b:b + 1, :] = zrow
            out_ref[b + SZ:b + XPADP, :] = zpad
            out_ref[b + 1:b + SZ + 1, 0:32] = pk0[r]
            out_ref[b:b + SZ, 32:64] = pk0[r]
            out_ref[b + 1:b + SZ + 1, 64:96] = pk1[r]
            out_ref[b:b + SZ, 96:128] = pk1[r]


def _build_tables(xy, xz, yz, wlo, whi):
    grid = YPAD // CHY
    last = SZ // CHY - 1
    bs_plane = pl.BlockSpec((DIM, CHY, SZ),
                            lambda j: (0, jnp.minimum(j, last), 0))
    bs_w = pl.BlockSpec((32, 3 * DIM), lambda j: (0, 0))
    bs_out = pl.BlockSpec((CHY * XPADP, 2 * DIM), lambda j: (j, 0))
    tab = jax.ShapeDtypeStruct((YPAD * XPADP, 2 * DIM), jnp.int32)
    return pl.pallas_call(
        _tables_body,
        grid=(grid,),
        in_specs=[bs_plane, bs_plane, bs_plane, bs_w, bs_w],
        out_specs=[bs_out, bs_out, bs_out],
        out_shape=[tab, tab, tab],
        scratch_shapes=[pltpu.VMEM((3, 2, SZ, 32), jnp.float32)],
    )(xy, xz, yz, wlo, whi)


def _axis_setup(g):
    ixf = ((g + 1.0) * float(SZ) - 1.0) * 0.5
    ixf = jnp.clip(ixf, -1.0, float(SZ))
    t = ixf.astype(jnp.int32)
    tf = t.astype(jnp.float32)
    c0 = jnp.where(tf > ixf, t - 1, t)
    w1 = ixf - c0.astype(jnp.float32)
    return c0, 1.0 - w1, w1


def _sc_sample(gx, gy, gz, tabs, bias):
    n = gx.shape[0]
    info = plsc.get_sparse_core_info()
    nw = info.num_cores * info.num_subcores
    npw = n // nw
    nblk = npw // P
    mesh = plsc.VectorSubcoreMesh(core_axis_name="c", subcore_axis_name="s")

    @functools.partial(
        pl.kernel, mesh=mesh,
        out_type=jax.ShapeDtypeStruct((n, DIM), jnp.float32),
        compiler_params=pltpu.CompilerParams(use_tc_tiling_on_sc=False,
                                             needs_layout_passes=False),
        scratch_types=[
            pltpu.VMEM((P,), jnp.float32),
            pltpu.VMEM((P,), jnp.float32),
            pltpu.VMEM((P,), jnp.float32),
            pltpu.VMEM((2, 3, P), jnp.int32),
            pltpu.VMEM((2, 12, P), jnp.float32),
            pltpu.VMEM((3, P, 2 * DIM), jnp.int32),
            pltpu.VMEM((3, P, 2 * DIM), jnp.int32),
            pltpu.VMEM((P, DIM), jnp.float32),
            pltpu.VMEM((P, DIM), jnp.float32),
            pltpu.VMEM((DIM,), jnp.float32),
            pltpu.SemaphoreType.DMA,
            pltpu.SemaphoreType.DMA,
            pltpu.SemaphoreType.DMA,
            pltpu.SemaphoreType.DMA,
        ],
    )
    def body(gx_h, gy_h, gz_h, t0_h, t1_h, t2_h, b_h, out_h,
             gx_v, gy_v, gz_v, idx_v, w_v, rows0_v, rows1_v, acc0_v, acc1_v,
             b_v, gsem0, gsem1, osem0, osem1):
        tab_hs = (t0_h, t1_h, t2_h)
        rows_vs = (rows0_v, rows1_v)
        acc_vs = (acc0_v, acc1_v)
        gsems = (gsem0, gsem1)
        osems = (osem0, osem1)
        wid = lax.axis_index("s") * info.num_cores + lax.axis_index("c")
        base0 = wid * npw
        pltpu.sync_copy(b_h, b_v)
        bias_regs = [b_v[pl.ds(k * 16, 16)] for k in range(4)]

        def stage_a(blk, buf):
            base = base0 + blk * P
            pltpu.sync_copy(gx_h.at[pl.ds(base, P)], gx_v)
            pltpu.sync_copy(gy_h.at[pl.ds(base, P)], gy_v)
            pltpu.sync_copy(gz_h.at[pl.ds(base, P)], gz_v)

            def grp_body(i, c2):
                s = pl.ds(i * 16, 16)
                ax_ = _axis_setup(gx_v[s])
                ay_ = _axis_setup(gy_v[s])
                az_ = _axis_setup(gz_v[s])
                for p, (axA, axB) in enumerate(((ax_, ay_), (ax_, az_), (ay_, az_))):
                    a0, aw0, aw1 = axA
                    b0, bw0, bw1 = axB
                    idx_v[buf, p, s] = (b0 + 1) * XPADP + a0 + 1
                    w_v[buf, 4 * p + 0, s] = aw0 * bw0
                    w_v[buf, 4 * p + 1, s] = aw1 * bw0
                    w_v[buf, 4 * p + 2, s] = aw0 * bw1
                    w_v[buf, 4 * p + 3, s] = aw1 * bw1
                return c2
            lax.fori_loop(0, P // 16, grp_body, 0)
            for r in range(3):
                pltpu.async_copy(tab_hs[r].at[idx_v.at[buf, r]],
                                 rows_vs[buf].at[r], gsems[buf])

        def wait_gathers(buf):
            for r in range(3):
                pltpu.make_async_copy(tab_hs[r].at[idx_v.at[buf, r]],
                                      rows_vs[buf].at[r], gsems[buf]).wait()

        def drain_out(buf):
            pltpu.make_async_copy(acc_vs[buf], out_h.at[pl.ds(base0, P)],
                                  osems[buf]).wait()

        def stage_b(blk, buf):
            rows_v = rows_vs[buf]
            acc_v = acc_vs[buf]

            def ptg_body(g, c2):
                sg = pl.ds(g * 16, 16)
                wvecs = [w_v[buf, c, sg] for c in range(12)]
                rbase = g * 16
                for j in range(16):
                    m = rbase + j
                    accs = list(bias_regs)
                    for p in range(3):
                        for c in range(4):
                            w = jnp.full((16,), wvecs[4 * p + c][j],
                                         jnp.float32)
                            for q in range(2):
                                v = rows_v[p, m, pl.ds(c * 32 + q * 16, 16)]
                                even = plsc.bitcast(v << 16, jnp.float32)
                                odd = plsc.bitcast(v, jnp.float32)
                                accs[2 * q] = accs[2 * q] + w * even
                                accs[2 * q + 1] = accs[2 * q + 1] + w * odd
                    for k in range(4):
                        acc_v[m, pl.ds(k * 16, 16)] = accs[k]
                return c2
            lax.fori_loop(0, P // 16, ptg_body, 0)
            pltpu.async_copy(acc_v, out_h.at[pl.ds(base0 + blk * P, P)],
                             osems[buf])

        stage_a(0, 0)
        stage_a(1, 1)

        def pair_body(jp, carry):
            for buf in range(2):
                blk = 2 * jp + buf
                wait_gathers(buf)

                @pl.when(jp > 0)
                def _():
                    drain_out(buf)
                stage_b(blk, buf)

                @pl.when(jp < nblk // 2 - 1)
                def _():
                    stage_a(blk + 2, buf)
            return carry
        lax.fori_loop(0, nblk // 2, pair_body, 0)
        drain_out(0)
        drain_out(1)

    return body(gx, gy, gz, *tabs, bias)


def kernel(x, xy, xz, yz, lin_w, lin_b):
    wlo = jnp.concatenate([lin_w[0:16], lin_w[32:48]], 0)
    whi = jnp.concatenate([lin_w[16:32], lin_w[48:64]], 0)
    tabs = _build_tables(xy, xz, yz, wlo, whi)
    gx, gy, gz = x[:, 0], x[:, 1], x[:, 2]
    return _sc_sample(gx, gy, gz, tabs, lin_b)


# speedup vs baseline: 1.1846x; 1.0522x over previous
import functools

import jax
import jax.numpy as jnp
import numpy as np
from jax import lax
from jax.experimental import pallas as pl
from jax.experimental.pallas import tpu as pltpu
from jax.experimental.pallas import tpu_sc as plsc

DIM = 64
SZ = 512
CHY = 8
XPADP = 520
YPAD = 520
P = 128

def _pack32(tlo, thi):
    ulo = lax.bitcast_convert_type(tlo, jnp.uint32) >> 16
    uhi = lax.bitcast_convert_type(thi, jnp.uint32) & jnp.uint32(0xFFFF0000)
    return lax.bitcast_convert_type(ulo | uhi, jnp.int32)


def _tables_body(xy_ref, xz_ref, yz_ref, wlo_ref, whi_ref,
                 o0_ref, o1_ref, o2_ref, carry_ref):
    j = pl.program_id(0)

    @pl.when(j == 0)
    def _():
        carry_ref[...] = jnp.zeros((3, 2, SZ, 32), jnp.float32)

    live = (j <= SZ // CHY - 1)
    for p, (ref, out_ref) in enumerate(((xy_ref, o0_ref), (xz_ref, o1_ref),
                                        (yz_ref, o2_ref))):
        a = ref[...]
        halves = []
        for h, w_ref in enumerate((wlo_ref, whi_ref)):
            wp = w_ref[:, p * DIM:(p + 1) * DIM]
            t = lax.dot_general(a, wp, (((0,), (1,)), ((), ())),
                                preferred_element_type=jnp.float32)
            t = jnp.where(live, t, 0.0)
            prev = carry_ref[p, h]
            halves.append((jnp.concatenate([prev[None], t[:-1]], 0), t))
            carry_ref[p, h] = t[-1]
        (lo_m1, lo), (hi_m1, hi) = halves
        pk0 = _pack32(lo_m1, hi_m1)
        pk1 = _pack32(lo, hi)
        pka = jnp.concatenate([pk0, pk1], -1)
        zpad = jnp.zeros((XPADP - SZ, 2 * DIM), jnp.int32)
        zrow = jnp.zeros((1, 2 * DIM), jnp.int32)
        for r in range(CHY):
            b = r * XPADP
            out_ref[b:b + 1, :] = zrow
            out_ref[b + SZ:b + XPADP, :] = zpad
            out_ref[b + 1:b + SZ + 1, 0:DIM] = pka[r]
            out_ref[b:b + SZ, DIM:2 * DIM] = pka[r]


def _build_tables(xy, xz, yz, wlo, whi):
    grid = YPAD // CHY
    last = SZ // CHY - 1
    bs_plane = pl.BlockSpec((DIM, CHY, SZ),
                            lambda j: (0, jnp.minimum(j, last), 0))
    bs_w = pl.BlockSpec((32, 3 * DIM), lambda j: (0, 0))
    bs_out = pl.BlockSpec((CHY * XPADP, 2 * DIM), lambda j: (j, 0))
    tab = jax.ShapeDtypeStruct((YPAD * XPADP, 2 * DIM), jnp.int32)
    return pl.pallas_call(
        _tables_body,
        grid=(grid,),
        in_specs=[bs_plane, bs_plane, bs_plane, bs_w, bs_w],
        out_specs=[bs_out, bs_out, bs_out],
        out_shape=[tab, tab, tab],
        scratch_shapes=[pltpu.VMEM((3, 2, SZ, 32), jnp.float32)],
    )(xy, xz, yz, wlo, whi)


def _axis_setup(g):
    ixf = ((g + 1.0) * float(SZ) - 1.0) * 0.5
    ixf = jnp.clip(ixf, -1.0, float(SZ))
    t = ixf.astype(jnp.int32)
    tf = t.astype(jnp.float32)
    c0 = jnp.where(tf > ixf, t - 1, t)
    w1 = ixf - c0.astype(jnp.float32)
    return c0, 1.0 - w1, w1


def _sc_sample(gx, gy, gz, tabs, bias):
    n = gx.shape[0]
    info = plsc.get_sparse_core_info()
    nw = info.num_cores * info.num_subcores
    npw = n // nw
    nblk = npw // P
    mesh = plsc.VectorSubcoreMesh(core_axis_name="c", subcore_axis_name="s")

    @functools.partial(
        pl.kernel, mesh=mesh,
        out_type=jax.ShapeDtypeStruct((n // 2, 2 * DIM), jnp.float32),
        compiler_params=pltpu.CompilerParams(use_tc_tiling_on_sc=False,
                                             needs_layout_passes=False),
        scratch_types=[
            pltpu.VMEM((P,), jnp.float32),
            pltpu.VMEM((P,), jnp.float32),
            pltpu.VMEM((P,), jnp.float32),
            pltpu.VMEM((2, 3, P), jnp.int32),
            pltpu.VMEM((2, 12, P), jnp.float32),
            pltpu.VMEM((3, P, 2 * DIM), jnp.int32),
            pltpu.VMEM((3, P, 2 * DIM), jnp.int32),
            pltpu.VMEM((P // 2, 2 * DIM), jnp.float32),
            pltpu.VMEM((P // 2, 2 * DIM), jnp.float32),
            pltpu.VMEM((DIM,), jnp.float32),
            pltpu.SemaphoreType.DMA,
            pltpu.SemaphoreType.DMA,
            pltpu.SemaphoreType.DMA,
            pltpu.SemaphoreType.DMA,
        ],
    )
    def body(gx_h, gy_h, gz_h, t0_h, t1_h, t2_h, b_h, out_h,
             gx_v, gy_v, gz_v, idx_v, w_v, rows0_v, rows1_v, acc0_v, acc1_v,
             b_v, gsem0, gsem1, osem0, osem1):
        tab_hs = (t0_h, t1_h, t2_h)
        rows_vs = (rows0_v, rows1_v)
        acc_vs = (acc0_v, acc1_v)
        gsems = (gsem0, gsem1)
        osems = (osem0, osem1)
        wid = lax.axis_index("s") * info.num_cores + lax.axis_index("c")
        base0 = wid * npw
        pltpu.sync_copy(b_h, b_v)
        bias_regs = [b_v[pl.ds(k * 16, 16)] for k in range(4)]

        def stage_a(blk, buf):
            base = base0 + blk * P
            pltpu.sync_copy(gx_h.at[pl.ds(base, P)], gx_v)
            pltpu.sync_copy(gy_h.at[pl.ds(base, P)], gy_v)
            pltpu.sync_copy(gz_h.at[pl.ds(base, P)], gz_v)

            def grp_body(i, c2):
                s = pl.ds(i * 16, 16)
                ax_ = _axis_setup(gx_v[s])
                ay_ = _axis_setup(gy_v[s])
                az_ = _axis_setup(gz_v[s])
                for p, (axA, axB) in enumerate(((ax_, ay_), (ax_, az_), (ay_, az_))):
                    a0, aw0, aw1 = axA
                    b0, bw0, bw1 = axB
                    idx_v[buf, p, s] = (b0 + 1) * XPADP + a0 + 1
                    w_v[buf, 4 * p + 0, s] = aw0 * bw0
                    w_v[buf, 4 * p + 1, s] = aw0 * bw1
                    w_v[buf, 4 * p + 2, s] = aw1 * bw0
                    w_v[buf, 4 * p + 3, s] = aw1 * bw1
                return c2
            lax.fori_loop(0, P // 16, grp_body, 0)
            for r in range(3):
                pltpu.async_copy(tab_hs[r].at[idx_v.at[buf, r]],
                                 rows_vs[buf].at[r], gsems[buf])

        def wait_gathers(buf):
            for r in range(3):
                pltpu.make_async_copy(tab_hs[r].at[idx_v.at[buf, r]],
                                      rows_vs[buf].at[r], gsems[buf]).wait()

        def drain_out(buf):
            pltpu.make_async_copy(acc_vs[buf], out_h.at[pl.ds(base0 // 2, P // 2)],
                                  osems[buf]).wait()

        def stage_b(blk, buf):
            rows_v = rows_vs[buf]
            acc_v = acc_vs[buf]

            def ptg_body(g, c2):
                sg = pl.ds(g * 16, 16)
                wvecs = [w_v[buf, c, sg] for c in range(12)]
                rbase = g * 16
                for j in range(16):
                    m = rbase + j
                    accs = list(bias_regs)
                    for p in range(3):
                        for c in range(4):
                            w = jnp.full((16,), wvecs[4 * p + c][j],
                                         jnp.float32)
                            for q in range(2):
                                v = rows_v[p, m, pl.ds(c * 32 + q * 16, 16)]
                                even = plsc.bitcast(v << 16, jnp.float32)
                                odd = plsc.bitcast(v, jnp.float32)
                                accs[2 * q] = accs[2 * q] + w * even
                                accs[2 * q + 1] = accs[2 * q + 1] + w * odd
                    m2 = g * 8 + j // 2
                    half = (j % 2) * DIM
                    for k in range(4):
                        acc_v[m2, pl.ds(half + k * 16, 16)] = accs[k]
                return c2
            lax.fori_loop(0, P // 16, ptg_body, 0)
            pltpu.async_copy(acc_v, out_h.at[pl.ds((base0 + blk * P) // 2, P // 2)],
                             osems[buf])

        stage_a(0, 0)
        stage_a(1, 1)

        def pair_body(jp, carry):
            for buf in range(2):
                blk = 2 * jp + buf
                wait_gathers(buf)

                @pl.when(jp > 0)
                def _():
                    drain_out(buf)
                stage_b(blk, buf)

                @pl.when(jp < nblk // 2 - 1)
                def _():
                    stage_a(blk + 2, buf)
            return carry
        lax.fori_loop(0, nblk // 2, pair_body, 0)
        drain_out(0)
        drain_out(1)

    return body(gx, gy, gz, *tabs, bias)


def kernel(x, xy, xz, yz, lin_w, lin_b):
    wlo = jnp.concatenate([lin_w[0:16], lin_w[32:48]], 0)
    whi = jnp.concatenate([lin_w[16:32], lin_w[48:64]], 0)
    tabs = _build_tables(xy, xz, yz, wlo, whi)
    gx, gy, gz = x[:, 0], x[:, 1], x[:, 2]
    n = x.shape[0]
    return _sc_sample(gx, gy, gz, tabs, lin_b).reshape(n, DIM)
